# baseline (speedup 1.0000x reference)
import jax
import jax.numpy as jnp
from jax import lax
from jax.experimental import pallas as pl
from jax.experimental.pallas import tpu as pltpu

K = 300
CAP = 384
Q = 900
C = 91
N = Q * C
HI0 = 0x7F800000
NB = 4
SUBQ = 112

bf16 = jnp.bfloat16
f32 = jnp.float32
i32 = jnp.int32


def _bmm(a, b):
    return lax.dot_general(a, b, (((1,), (0,)), ((), ())),
                           preferred_element_type=f32)


def _hmm(a, b):
    return lax.dot_general(a, b, (((1,), (0,)), ((), ())),
                           precision=lax.Precision.HIGHEST,
                           preferred_element_type=f32)


def _any(flags):
    a = flags[0]
    for x in flags[1:]:
        a = jnp.logical_or(a, x)
    return a


def _block_kernel(logits_ref, boxes_ref, sizes_ref,
                  scores_ref, labels_ref, boxes_out_ref):
    flat = (lax.broadcasted_iota(i32, (Q, C), 0) * C
            + lax.broadcasted_iota(i32, (Q, C), 1))

    kbs = []
    for r in range(NB):
        p = jax.nn.sigmoid(logits_ref[r])
        kbs.append(lax.bitcast_convert_type(p, i32))

    def count_gt(r, t):
        return jnp.sum((kbs[r] > t).astype(i32))

    def seed_cond(st):
        los, his, dones = st
        return _any([jnp.logical_and(jnp.logical_not(dones[r]),
                                     his[r] - los[r] > 1) for r in range(NB)])

    def seed_body(st):
        los, his, dones = st
        nlo, nhi, ndone = [], [], []
        for r in range(NB):
            mid = los[r] + (his[r] - los[r]) // 2
            c = jnp.sum((kbs[r][:SUBQ] > mid).astype(i32))
            go_lo = c > 42
            nlo.append(jnp.where(go_lo, mid, los[r]))
            nhi.append(jnp.where(go_lo, his[r], mid))
            ndone.append(jnp.logical_or(dones[r],
                                        jnp.logical_and(c >= 30, c <= 42)))
        return (nlo, nhi, ndone)

    z, inf = jnp.int32(-1), jnp.int32(HI0)
    slos, shis, _ = lax.while_loop(
        seed_cond, seed_body,
        ([z] * NB, [inf] * NB, [jnp.bool_(False)] * NB))

    def ph1_cond(st):
        step, los, his, tks, cs, founds = st
        return _any([jnp.logical_and(jnp.logical_not(founds[r]),
                                     his[r] - los[r] > 1) for r in range(NB)])

    def ph1_body(st):
        step, los, his, tks, cs, founds = st
        nlo, nhi, ntk, ncs, nfound = [], [], [], [], []
        for r in range(NB):
            bis = los[r] + (his[r] - los[r]) // 2
            mid = jnp.where(step == 0, shis[r],
                            jnp.where(step == 1, slos[r], bis))
            mid = jnp.clip(mid, los[r] + 1, his[r] - 1)
            c = count_gt(r, mid)
            ok = jnp.logical_and(c >= K, c <= CAP)
            upd = jnp.logical_not(founds[r])
            nlo.append(jnp.where(jnp.logical_and(upd, c > CAP), mid, los[r]))
            nhi.append(jnp.where(jnp.logical_and(upd, c < K), mid, his[r]))
            ntk.append(jnp.where(jnp.logical_and(upd, ok), mid, tks[r]))
            ncs.append(jnp.where(jnp.logical_and(upd, ok), c, cs[r]))
            nfound.append(jnp.logical_or(founds[r], ok))
        return (step + 1, nlo, nhi, ntk, ncs, nfound)

    zero = jnp.int32(0)
    _, los, his, tks, cs, found1 = lax.while_loop(
        ph1_cond, ph1_body,
        (zero, [z] * NB, [inf] * NB, [zero] * NB, [zero] * NB,
         [jnp.bool_(False)] * NB))

    tks = [jnp.where(found1[r], tks[r], his[r]) for r in range(NB)]
    eqms = [kbs[r] == tks[r] for r in range(NB)]

    def ph2_cond(st):
        los_i, his_i, tis, cs2, founds = st
        return _any([jnp.logical_and(jnp.logical_not(founds[r]),
                                     his_i[r] - los_i[r] > 1)
                     for r in range(NB)])

    def ph2_body(st):
        los_i, his_i, tis, cs2, founds = st
        nlo, nhi, nti, ncs, nfound = [], [], [], [], []
        for r in range(NB):
            mid = los_i[r] + (his_i[r] - los_i[r]) // 2
            c = (count_gt(r, tks[r])
                 + jnp.sum(jnp.logical_and(eqms[r], flat < mid).astype(i32)))
            ok = jnp.logical_and(c >= K, c <= CAP)
            upd = jnp.logical_not(founds[r])
            nlo.append(jnp.where(jnp.logical_and(upd, c < K), mid, los_i[r]))
            nhi.append(jnp.where(jnp.logical_and(upd, c > CAP), mid,
                                 his_i[r]))
            nti.append(jnp.where(jnp.logical_and(upd, ok), mid, tis[r]))
            ncs.append(jnp.where(jnp.logical_and(upd, ok), c, cs2[r]))
            nfound.append(jnp.logical_or(founds[r], ok))
        return (nlo, nhi, nti, ncs, nfound)

    npad = jnp.int32(N)
    _, _, tis, cs, _ = lax.while_loop(
        ph2_cond, ph2_body,
        ([zero] * NB, [npad] * NB, [zero] * NB, cs, found1))

    li = lax.broadcasted_iota(i32, (C, C), 0)
    lj = lax.broadcasted_iota(i32, (C, C), 1)
    upper = (li < lj).astype(bf16)
    ri = lax.broadcasted_iota(i32, (Q, Q), 0)
    rj = lax.broadcasted_iota(i32, (Q, Q), 1)
    ltri = (rj < ri).astype(bf16)
    s_iota = lax.broadcasted_iota(i32, (CAP, 1), 0).astype(f32)
    lane_iota = lax.broadcasted_iota(i32, (CAP, C), 1).astype(f32)
    q_lane = lax.broadcasted_iota(i32, (CAP, Q), 1).astype(f32)
    cap_iota = lax.broadcasted_iota(i32, (CAP, CAP), 0).astype(f32)
    row_f = lax.broadcasted_iota(i32, (Q, 1), 0)
    row_hi = (row_f >> 8).astype(f32)
    row_lo = (row_f & 0xFF).astype(f32)

    for r in range(NB):
        kb = kbs[r]
        sel = jnp.logical_or(kb > tks[r],
                             jnp.logical_and(eqms[r], flat < tis[r]))
        maskf = sel.astype(f32)
        cstar = cs[r].astype(f32)

        pos = _bmm(maskf.astype(bf16), upper)
        cnt = pos[:, C - 1:C] + maskf[:, C - 1:C]
        offs = _bmm(ltri, cnt.astype(bf16))

        oc_t = jnp.concatenate([offs, cnt], axis=1).T
        offs_t = oc_t[0:1]
        cnt_t = oc_t[1:2]
        Rsel = jnp.logical_and(offs_t <= s_iota,
                               s_iota < offs_t + cnt_t)

        kbf0 = (kb & 0xFF).astype(f32)
        kbf1 = ((kb >> 8) & 0xFF).astype(f32)
        kbf2 = ((kb >> 16) & 0xFF).astype(f32)
        kbf3 = ((kb >> 24) & 0x7F).astype(f32)
        posm = 2.0 * pos + maskf
        rhs = jnp.concatenate(
            [kbf0, kbf1, kbf2, kbf3, posm,
             (offs >= 256.0).astype(f32),
             offs - 256.0 * (offs >= 256.0).astype(f32),
             row_hi, row_lo], axis=1).astype(bf16)
        G = _bmm(Rsel.astype(bf16), rhs)
        g0 = G[:, 0 * C:1 * C]
        g1 = G[:, 1 * C:2 * C]
        g2 = G[:, 2 * C:3 * C]
        g3 = G[:, 3 * C:4 * C]
        gposm = G[:, 4 * C:5 * C]
        base = 5 * C
        offs_sel = 256.0 * G[:, base:base + 1] + G[:, base + 1:base + 2]
        r_sel = 256.0 * G[:, base + 2:base + 3] + G[:, base + 3:base + 4]

        Lf = (gposm == 2.0 * (s_iota - offs_sel) + 1.0).astype(f32)
        b0 = jnp.sum(g0 * Lf, axis=1, keepdims=True)
        b1 = jnp.sum(g1 * Lf, axis=1, keepdims=True)
        b2 = jnp.sum(g2 * Lf, axis=1, keepdims=True)
        b3 = jnp.sum(g3 * Lf, axis=1, keepdims=True)
        lane = jnp.sum(Lf * lane_iota, axis=1, keepdims=True)
        flatc = r_sel * C + lane

        hi16 = 256.0 * b3 + b2
        lo16 = 256.0 * b1 + b0
        val = lax.bitcast_convert_type(
            (hi16.astype(i32) << 16) | lo16.astype(i32), f32)
        valid = s_iota < cstar
        hi16 = jnp.where(valid, hi16, -1.0)
        lo16 = jnp.where(valid, lo16, -1.0)
        flatc = jnp.where(valid, flatc, 1e9)

        labelf = lane
        Bx = (r_sel == q_lane).astype(bf16)
        bx = boxes_ref[r]
        bx_hi = bx.astype(bf16)
        bx_lo = (bx - bx_hi.astype(f32)).astype(bf16)
        bg = _bmm(Bx, jnp.concatenate([bx_hi, bx_lo], axis=1))
        boxcand = bg[:, 0:4] + bg[:, 4:8]

        hlf_t = jnp.concatenate([hi16, lo16, flatc], axis=1).T
        hi_t, lo_t, fl_t = hlf_t[0:1], hlf_t[1:2], hlf_t[2:3]
        beats = jnp.logical_or(
            hi_t > hi16,
            jnp.logical_and(
                hi_t == hi16,
                jnp.logical_or(lo_t > lo16,
                               jnp.logical_and(lo_t == lo16,
                                               fl_t < flatc))))
        rank = jnp.sum(beats.astype(f32), axis=1, keepdims=True)
        Sp = (rank.T == cap_iota).astype(f32)

        feats = jnp.concatenate([val, labelf, boxcand], axis=1)
        sorted_f = _hmm(Sp, feats)[:K]

        scores = sorted_f[:, 0:1]
        labels = sorted_f[:, 1:2]
        cx = sorted_f[:, 2:3]
        cy = sorted_f[:, 3:4]
        w = sorted_f[:, 4:5]
        h = sorted_f[:, 5:6]
        xyxy = jnp.concatenate(
            [cx - 0.5 * w, cy - 0.5 * h, cx + 0.5 * w, cy + 0.5 * h], axis=1)
        sz = sizes_ref[r].astype(f32)
        img_h = sz[0:1, 0:1]
        img_w = sz[0:1, 1:2]
        scale = jnp.concatenate([img_w, img_h, img_w, img_h], axis=1)
        boxes_out_ref[r] = xyxy * scale
        scores_ref[r] = scores.T
        labels_ref[r] = jnp.round(labels).astype(i32).T


def kernel(pred_logits, pred_boxes, orig_sizes):
    B = pred_logits.shape[0]
    sizes = orig_sizes.reshape(B, 1, 2)

    scores, labels, boxes = pl.pallas_call(
        _block_kernel,
        grid=(B // NB,),
        in_specs=[
            pl.BlockSpec((NB, Q, C), lambda i: (i, 0, 0)),
            pl.BlockSpec((NB, Q, 4), lambda i: (i, 0, 0)),
            pl.BlockSpec((NB, 1, 2), lambda i: (i, 0, 0)),
        ],
        out_specs=[
            pl.BlockSpec((NB, 1, K), lambda i: (i, 0, 0)),
            pl.BlockSpec((NB, 1, K), lambda i: (i, 0, 0)),
            pl.BlockSpec((NB, K, 4), lambda i: (i, 0, 0)),
        ],
        out_shape=[
            jax.ShapeDtypeStruct((B, 1, K), f32),
            jax.ShapeDtypeStruct((B, 1, K), i32),
            jax.ShapeDtypeStruct((B, K, 4), f32),
        ],
        compiler_params=pltpu.CompilerParams(
            dimension_semantics=("parallel",)),
    )(pred_logits, pred_boxes, sizes)

    return scores.reshape(B, K), labels.reshape(B, K), boxes


# speedup vs baseline: 1.2332x; 1.2332x over previous
import jax
import jax.numpy as jnp
from jax import lax
from jax.experimental import pallas as pl
from jax.experimental.pallas import tpu as pltpu

K = 300
CAP = 384
Q = 900
C = 91
N = Q * C
NPAD = 81920
ROWS = 640
LANES = 128
HI0 = 0x7F800000
NB = 8

bf16 = jnp.bfloat16
f32 = jnp.float32
i32 = jnp.int32


def _bmm(a, b):
    return lax.dot_general(a, b, (((1,), (0,)), ((), ())),
                           preferred_element_type=f32)


def _hmm(a, b):
    return lax.dot_general(a, b, (((1,), (0,)), ((), ())),
                           precision=lax.Precision.HIGHEST,
                           preferred_element_type=f32)


def _any(flags):
    a = flags[0]
    for x in flags[1:]:
        a = jnp.logical_or(a, x)
    return a


def _block_kernel(logits_ref, boxes_ref, scale_ref,
                  scores_ref, labels_ref, boxes_out_ref):
    flat = (lax.broadcasted_iota(i32, (ROWS, LANES), 0) * LANES
            + lax.broadcasted_iota(i32, (ROWS, LANES), 1))
    pad_ok = flat < N

    ps, kbs = [], []
    for r in range(NB):
        p = jax.nn.sigmoid(logits_ref[r])
        p = jnp.where(pad_ok, p, -1.0)
        ps.append(p)
        kbs.append(lax.bitcast_convert_type(p, i32))

    def count_gt(r, t):
        return jnp.sum((kbs[r] > t).astype(i32))

    def seed_cond(st):
        los, his, dones = st
        return _any([jnp.logical_and(jnp.logical_not(dones[r]),
                                     his[r] - los[r] > 1) for r in range(NB)])

    def seed_body(st):
        los, his, dones = st
        nlo, nhi, ndone = [], [], []
        for r in range(NB):
            mid = los[r] + (his[r] - los[r]) // 2
            c = jnp.sum((kbs[r][:80] > mid).astype(i32))
            go_lo = c > 42
            nlo.append(jnp.where(go_lo, mid, los[r]))
            nhi.append(jnp.where(go_lo, his[r], mid))
            ndone.append(jnp.logical_or(dones[r],
                                        jnp.logical_and(c >= 30, c <= 42)))
        return (nlo, nhi, ndone)

    z, inf = jnp.int32(-1), jnp.int32(HI0)
    slos, shis, _ = lax.while_loop(
        seed_cond, seed_body,
        ([z] * NB, [inf] * NB, [jnp.bool_(False)] * NB))

    def ph1_cond(st):
        step, los, his, tks, cs, founds = st
        return _any([jnp.logical_and(jnp.logical_not(founds[r]),
                                     his[r] - los[r] > 1) for r in range(NB)])

    def ph1_body(st):
        step, los, his, tks, cs, founds = st
        nlo, nhi, ntk, ncs, nfound = [], [], [], [], []
        for r in range(NB):
            bis = los[r] + (his[r] - los[r]) // 2
            mid = jnp.where(step == 0, shis[r],
                            jnp.where(step == 1, slos[r], bis))
            mid = jnp.clip(mid, los[r] + 1, his[r] - 1)
            c = count_gt(r, mid)
            ok = jnp.logical_and(c >= K, c <= CAP)
            upd = jnp.logical_not(founds[r])
            nlo.append(jnp.where(jnp.logical_and(upd, c > CAP), mid, los[r]))
            nhi.append(jnp.where(jnp.logical_and(upd, c < K), mid, his[r]))
            ntk.append(jnp.where(jnp.logical_and(upd, ok), mid, tks[r]))
            ncs.append(jnp.where(jnp.logical_and(upd, ok), c, cs[r]))
            nfound.append(jnp.logical_or(founds[r], ok))
        return (step + 1, nlo, nhi, ntk, ncs, nfound)

    zero = jnp.int32(0)
    _, los, his, tks, cs, found1 = lax.while_loop(
        ph1_cond, ph1_body,
        (zero, [z] * NB, [inf] * NB, [zero] * NB, [zero] * NB,
         [jnp.bool_(False)] * NB))

    tks = [jnp.where(found1[r], tks[r], his[r]) for r in range(NB)]
    eqms = [kbs[r] == tks[r] for r in range(NB)]

    def ph2_cond(st):
        los_i, his_i, tis, cs2, founds = st
        return _any([jnp.logical_and(jnp.logical_not(founds[r]),
                                     his_i[r] - los_i[r] > 1)
                     for r in range(NB)])

    def ph2_body(st):
        los_i, his_i, tis, cs2, founds = st
        nlo, nhi, nti, ncs, nfound = [], [], [], [], []
        for r in range(NB):
            mid = los_i[r] + (his_i[r] - los_i[r]) // 2
            c = (count_gt(r, tks[r])
                 + jnp.sum(jnp.logical_and(eqms[r], flat < mid).astype(i32)))
            ok = jnp.logical_and(c >= K, c <= CAP)
            upd = jnp.logical_not(founds[r])
            nlo.append(jnp.where(jnp.logical_and(upd, c < K), mid, los_i[r]))
            nhi.append(jnp.where(jnp.logical_and(upd, c > CAP), mid,
                                 his_i[r]))
            nti.append(jnp.where(jnp.logical_and(upd, ok), mid, tis[r]))
            ncs.append(jnp.where(jnp.logical_and(upd, ok), c, cs2[r]))
            nfound.append(jnp.logical_or(founds[r], ok))
        return (nlo, nhi, nti, ncs, nfound)

    npad = jnp.int32(NPAD)
    _, _, tis, cs, _ = lax.while_loop(
        ph2_cond, ph2_body,
        ([zero] * NB, [npad] * NB, [zero] * NB, cs, found1))

    li = lax.broadcasted_iota(i32, (LANES, LANES), 0)
    lj = lax.broadcasted_iota(i32, (LANES, LANES), 1)
    upper = (li < lj).astype(bf16)
    ri = lax.broadcasted_iota(i32, (ROWS, ROWS), 0)
    rj = lax.broadcasted_iota(i32, (ROWS, ROWS), 1)
    ltri = (rj < ri).astype(bf16)
    s_iota = lax.broadcasted_iota(i32, (CAP, 1), 0).astype(f32)
    lane_iota = lax.broadcasted_iota(i32, (CAP, LANES), 1).astype(f32)
    q_lane = lax.broadcasted_iota(i32, (CAP, Q), 1).astype(f32)
    cap_iota = lax.broadcasted_iota(i32, (CAP, CAP), 0).astype(f32)
    row_f = lax.broadcasted_iota(i32, (ROWS, 1), 0)
    row_hi = (row_f >> 8).astype(f32)
    row_lo = (row_f & 0xFF).astype(f32)

    for r in range(NB):
        kb = kbs[r]
        sel = jnp.logical_or(kb > tks[r],
                             jnp.logical_and(eqms[r], flat < tis[r]))
        maskf = sel.astype(f32)
        cstar = cs[r].astype(f32)

        pos = _bmm(maskf.astype(bf16), upper)
        cnt = pos[:, LANES - 1:LANES] + maskf[:, LANES - 1:LANES]
        offs = _bmm(ltri, cnt.astype(bf16))

        oc_t = jnp.concatenate([offs, cnt], axis=1).T
        offs_t = oc_t[0:1]
        cnt_t = oc_t[1:2]
        Rsel = jnp.logical_and(offs_t <= s_iota,
                               s_iota < offs_t + cnt_t)

        kbf0 = (kb & 0xFF).astype(f32)
        kbf1 = ((kb >> 8) & 0xFF).astype(f32)
        kbf2 = ((kb >> 16) & 0xFF).astype(f32)
        kbf3 = ((kb >> 24) & 0x7F).astype(f32)
        posm = 2.0 * pos + maskf
        rhs = jnp.concatenate(
            [kbf0, kbf1, kbf2, kbf3, posm,
             (offs >= 256.0).astype(f32),
             offs - 256.0 * (offs >= 256.0).astype(f32),
             row_hi, row_lo], axis=1).astype(bf16)
        G = _bmm(Rsel.astype(bf16), rhs)
        g0 = G[:, 0 * LANES:1 * LANES]
        g1 = G[:, 1 * LANES:2 * LANES]
        g2 = G[:, 2 * LANES:3 * LANES]
        g3 = G[:, 3 * LANES:4 * LANES]
        gposm = G[:, 4 * LANES:5 * LANES]
        base = 5 * LANES
        offs_sel = 256.0 * G[:, base:base + 1] + G[:, base + 1:base + 2]
        r_sel = 256.0 * G[:, base + 2:base + 3] + G[:, base + 3:base + 4]

        Lf = (gposm == 2.0 * (s_iota - offs_sel) + 1.0).astype(f32)
        b0 = jnp.sum(g0 * Lf, axis=1, keepdims=True)
        b1 = jnp.sum(g1 * Lf, axis=1, keepdims=True)
        b2 = jnp.sum(g2 * Lf, axis=1, keepdims=True)
        b3 = jnp.sum(g3 * Lf, axis=1, keepdims=True)
        lane = jnp.sum(Lf * lane_iota, axis=1, keepdims=True)
        flatc = r_sel * LANES + lane

        hi16 = 256.0 * b3 + b2
        lo16 = 256.0 * b1 + b0
        val = lax.bitcast_convert_type(
            (hi16.astype(i32) << 16) | lo16.astype(i32), f32)
        valid = s_iota < cstar
        hi16 = jnp.where(valid, hi16, -1.0)
        lo16 = jnp.where(valid, lo16, -1.0)
        flatc = jnp.where(valid, flatc, 1e9)

        qf = jnp.floor(flatc * (1.0 / C) + 5e-4)
        labelf = flatc - C * qf

        Bx = (qf == q_lane).astype(bf16)
        bx = boxes_ref[r]
        bx_hi = bx.astype(bf16)
        bx_lo = (bx - bx_hi.astype(f32)).astype(bf16)
        bg = _bmm(Bx, jnp.concatenate([bx_hi, bx_lo], axis=1))
        boxcand = bg[:, 0:4] + bg[:, 4:8]

        hlf_t = jnp.concatenate([hi16, lo16, flatc], axis=1).T
        hi_t, lo_t, fl_t = hlf_t[0:1], hlf_t[1:2], hlf_t[2:3]
        beats = jnp.logical_or(
            hi_t > hi16,
            jnp.logical_and(
                hi_t == hi16,
                jnp.logical_or(lo_t > lo16,
                               jnp.logical_and(lo_t == lo16,
                                               fl_t < flatc))))
        rank = jnp.sum(beats.astype(f32), axis=1, keepdims=True)
        Sp = (rank.T == cap_iota).astype(f32)

        feats = jnp.concatenate([val, labelf, boxcand], axis=1)
        sorted_f = _hmm(Sp, feats)[:K]

        scores = sorted_f[:, 0:1]
        labels = sorted_f[:, 1:2]
        cx = sorted_f[:, 2:3]
        cy = sorted_f[:, 3:4]
        w = sorted_f[:, 4:5]
        h = sorted_f[:, 5:6]
        xyxy = jnp.concatenate(
            [cx - 0.5 * w, cy - 0.5 * h, cx + 0.5 * w, cy + 0.5 * h], axis=1)
        boxes_out_ref[r] = xyxy * scale_ref[r]
        scores_ref[r] = scores.T
        labels_ref[r] = jnp.round(labels).astype(i32).T


def kernel(pred_logits, pred_boxes, orig_sizes):
    B = pred_logits.shape[0]
    logits = jnp.pad(pred_logits.reshape(B, N), ((0, 0), (0, NPAD - N)))
    logits = logits.reshape(B, ROWS, LANES)
    img_h = orig_sizes[:, 0].astype(f32)
    img_w = orig_sizes[:, 1].astype(f32)
    scale = jnp.stack([img_w, img_h, img_w, img_h], axis=1).reshape(B, 1, 4)

    scores, labels, boxes = pl.pallas_call(
        _block_kernel,
        grid=(B // NB,),
        in_specs=[
            pl.BlockSpec((NB, ROWS, LANES), lambda i: (i, 0, 0)),
            pl.BlockSpec((NB, Q, 4), lambda i: (i, 0, 0)),
            pl.BlockSpec((NB, 1, 4), lambda i: (i, 0, 0)),
        ],
        out_specs=[
            pl.BlockSpec((NB, 1, K), lambda i: (i, 0, 0)),
            pl.BlockSpec((NB, 1, K), lambda i: (i, 0, 0)),
            pl.BlockSpec((NB, K, 4), lambda i: (i, 0, 0)),
        ],
        out_shape=[
            jax.ShapeDtypeStruct((B, 1, K), f32),
            jax.ShapeDtypeStruct((B, 1, K), i32),
            jax.ShapeDtypeStruct((B, K, 4), f32),
        ],
        compiler_params=pltpu.CompilerParams(
            dimension_semantics=("parallel",)),
    )(logits, pred_boxes, scale)

    return scores.reshape(B, K), labels.reshape(B, K), boxes
